# baseline (speedup 1.0000x reference)
import jax
import jax.numpy as jnp
from jax import lax
from jax.experimental import pallas as pl
from jax.experimental.pallas import tpu as pltpu
from jax.experimental.pallas import tpu_sc as plsc

VOCAB = 1000000
DIM = 64
B = 16384
CTX = 10
NEG = 5
TN = NEG + 1
W = 128
V2 = VOCAB // 2

NC = 2
NS = 16
NW = NC * NS
BPW = B // NW
C = 16
NCHUNK = BPW // C
NPAIR = NCHUNK // 2
CC = C * CTX
CT = C * TN


def _tree_sum(vals):
    while len(vals) > 1:
        nxt = [vals[i] + vals[i + 1] for i in range(0, len(vals) - 1, 2)]
        if len(vals) % 2:
            nxt.append(vals[-1])
        vals = nxt
    return vals[0]


def _sc_body(ctx_row_hbm, ctx_par_hbm, tn_row_hbm, tn_par_hbm,
             ctx_tab_hbm, tgt_tab_hbm,
             out_pos_hbm, out_neg_hbm,
             ctx_row_v, ctx_par_v, tn_row_v, tn_par_v,
             ctx_a, ctx_b, tn_a, tn_b, pos_v, neg_v, sem_a, sem_b):
    wid = lax.axis_index("s") * NC + lax.axis_index("c")
    base = wid * BPW

    pltpu.sync_copy(ctx_row_hbm.at[pl.ds(base * CTX, BPW * CTX)], ctx_row_v)
    pltpu.sync_copy(ctx_par_hbm.at[pl.ds(base * CTX, BPW * CTX)], ctx_par_v)
    pltpu.sync_copy(tn_row_hbm.at[pl.ds(base * TN, BPW * TN)], tn_row_v)
    pltpu.sync_copy(tn_par_hbm.at[pl.ds(base * TN, BPW * TN)], tn_par_v)

    iota16 = lax.iota(jnp.int32, 16)

    def fire(it, ctx_buf, tn_buf, sem):
        pltpu.async_copy(
            ctx_tab_hbm.at[ctx_row_v.at[pl.ds(it * CC, 128)]],
            ctx_buf.at[pl.ds(0, 128)], sem)
        pltpu.async_copy(
            ctx_tab_hbm.at[ctx_row_v.at[pl.ds(it * CC + 128, CC - 128)]],
            ctx_buf.at[pl.ds(128, CC - 128)], sem)
        pltpu.async_copy(
            tgt_tab_hbm.at[tn_row_v.at[pl.ds(it * CT, CT)]], tn_buf, sem)

    def wait(ctx_buf, tn_buf, sem):
        pltpu.make_async_copy(ctx_tab_hbm.at[pl.ds(0, CC)], ctx_buf,
                              sem).wait()
        pltpu.make_async_copy(tgt_tab_hbm.at[pl.ds(0, CT)], tn_buf,
                              sem).wait()

    def compute(it, ctx_buf, tn_buf):
        e = iota16
        ctx_r = [e * CTX + j for j in range(CTX)]
        ctx_p = [plsc.load_gather(ctx_par_v, [e * CTX + (it * CC + j)])
                 for j in range(CTX)]
        tn_r = [e * TN + k for k in range(TN)]
        tn_p = [plsc.load_gather(tn_par_v, [e * TN + (it * CT + k)])
                for k in range(TN)]

        zero = jnp.zeros((16,), jnp.float32)

        @plsc.parallel_loop(0, DIM, carry=(zero, (zero,) * NEG), unroll=4)
        def d_loop(d, carry):
            pos, negs = carry
            col = jnp.full((16,), d, jnp.int32)
            m = _tree_sum([plsc.load_gather(ctx_buf, [ctx_r[j],
                                                      ctx_p[j] + col])
                           for j in range(CTX)]) * (1.0 / CTX)
            t = plsc.load_gather(tn_buf, [tn_r[0], tn_p[0] + col])
            pos = pos + t * m
            negs = tuple(
                negs[k] + plsc.load_gather(
                    tn_buf, [tn_r[k + 1], tn_p[k + 1] + col]) * m
                for k in range(NEG))
            return pos, negs

        pos, negs = d_loop
        plsc.store_scatter(pos_v, [it * C + iota16], pos)
        for k in range(NEG):
            plsc.store_scatter(neg_v, [k * BPW + it * C + iota16], negs[k])

    fire(0, ctx_a, tn_a, sem_a)

    def pair_body(i, carry):
        it0 = 2 * i
        fire(it0 + 1, ctx_b, tn_b, sem_b)
        wait(ctx_a, tn_a, sem_a)
        compute(it0, ctx_a, tn_a)

        @pl.when(i < NPAIR - 1)
        def _():
            fire(it0 + 2, ctx_a, tn_a, sem_a)

        wait(ctx_b, tn_b, sem_b)
        compute(it0 + 1, ctx_b, tn_b)
        return carry

    lax.fori_loop(0, NPAIR, pair_body, 0)

    pltpu.sync_copy(pos_v, out_pos_hbm.at[pl.ds(base, BPW)])
    for k in range(NEG):
        pltpu.sync_copy(neg_v.at[pl.ds(k * BPW, BPW)],
                        out_neg_hbm.at[pl.ds(k * B + base, BPW)])


def _sc_scores(ctx_row, ctx_par, tn_row, tn_par, ctx_tab2, tgt_tab2):
    mesh = plsc.VectorSubcoreMesh(core_axis_name="c", subcore_axis_name="s")
    return pl.kernel(
        _sc_body,
        out_type=(jax.ShapeDtypeStruct((B,), jnp.float32),
                  jax.ShapeDtypeStruct((NEG * B,), jnp.float32)),
        mesh=mesh,
        compiler_params=pltpu.CompilerParams(needs_layout_passes=False),
        scratch_types=[
            pltpu.VMEM((BPW * CTX,), jnp.int32),
            pltpu.VMEM((BPW * CTX,), jnp.int32),
            pltpu.VMEM((BPW * TN,), jnp.int32),
            pltpu.VMEM((BPW * TN,), jnp.int32),
            pltpu.VMEM((CC, W), jnp.float32),
            pltpu.VMEM((CC, W), jnp.float32),
            pltpu.VMEM((CT, W), jnp.float32),
            pltpu.VMEM((CT, W), jnp.float32),
            pltpu.VMEM((BPW,), jnp.float32),
            pltpu.VMEM((NEG * BPW,), jnp.float32),
            pltpu.SemaphoreType.DMA,
            pltpu.SemaphoreType.DMA,
        ],
    )(ctx_row, ctx_par, tn_row, tn_par, ctx_tab2, tgt_tab2)


def _loss_body(pos_ref, neg_ref, out_ref):
    s = jnp.sum(jax.nn.log_sigmoid(pos_ref[...]))
    s = s + jnp.sum(jax.nn.log_sigmoid(-neg_ref[...]))
    out_ref[0, 0] = -s / B


def _tc_loss(pos2d, neg2d):
    return pl.pallas_call(
        _loss_body,
        out_shape=jax.ShapeDtypeStruct((1, 1), jnp.float32),
        out_specs=pl.BlockSpec(memory_space=pltpu.SMEM),
    )(pos2d, neg2d)


def kernel(context_indices, target_indices, negative_indices, context_table,
           target_table):
    ctx_idx = context_indices.astype(jnp.int32).reshape(B * CTX)
    tn_idx = jnp.concatenate(
        [target_indices.astype(jnp.int32).reshape(B, 1),
         negative_indices.astype(jnp.int32)], axis=1).reshape(B * TN)
    ctx_row = ctx_idx >> 1
    ctx_par = (ctx_idx & 1) * DIM
    tn_row = tn_idx >> 1
    tn_par = (tn_idx & 1) * DIM
    ctx_tab2 = context_table.reshape(V2, W)
    tgt_tab2 = target_table.reshape(V2, W)
    pos, neg = _sc_scores(ctx_row, ctx_par, tn_row, tn_par, ctx_tab2,
                          tgt_tab2)
    loss = _tc_loss(pos.reshape(128, 128), neg.reshape(NEG * 128, 128))
    return loss[0, 0]


# speedup vs baseline: 1.1898x; 1.1898x over previous
import jax
import jax.numpy as jnp
from jax import lax
from jax.experimental import pallas as pl
from jax.experimental.pallas import tpu as pltpu
from jax.experimental.pallas import tpu_sc as plsc

VOCAB = 1000000
DIM = 64
B = 16384
CTX = 10
NEG = 5
TN = NEG + 1
W = 128
V2 = VOCAB // 2

NC = 2
NS = 16
NW = NC * NS
BPW = B // NW
C = 16
NCHUNK = BPW // C
NPAIR = NCHUNK // 2
CC = C * CTX
CT = C * TN


def _tree_sum(vals):
    while len(vals) > 1:
        nxt = [vals[i] + vals[i + 1] for i in range(0, len(vals) - 1, 2)]
        if len(vals) % 2:
            nxt.append(vals[-1])
        vals = nxt
    return vals[0]


def _sc_body(ctx_row_hbm, ctx_par_hbm, tn_row_hbm, tn_par_hbm,
             ctx_tab_hbm, tgt_tab_hbm,
             out_pos_hbm, out_neg_hbm,
             ctx_row_v, ctx_par_v, tn_row_v, tn_par_v,
             ctx_a, ctx_b, tn_a, tn_b, pos_v, neg_v, sem_a, sem_b):
    wid = lax.axis_index("s") * NC + lax.axis_index("c")
    base = wid * BPW

    pltpu.sync_copy(ctx_row_hbm.at[pl.ds(base * CTX, BPW * CTX)], ctx_row_v)
    pltpu.sync_copy(ctx_par_hbm.at[pl.ds(base * CTX, BPW * CTX)], ctx_par_v)
    pltpu.sync_copy(tn_row_hbm.at[pl.ds(base * TN, BPW * TN)], tn_row_v)
    pltpu.sync_copy(tn_par_hbm.at[pl.ds(base * TN, BPW * TN)], tn_par_v)

    iota16 = lax.iota(jnp.int32, 16)

    def fire(it, ctx_buf, tn_buf, sem):
        pltpu.async_copy(
            ctx_tab_hbm.at[ctx_row_v.at[pl.ds(it * CC, 128)]],
            ctx_buf.at[pl.ds(0, 128)], sem)
        pltpu.async_copy(
            ctx_tab_hbm.at[ctx_row_v.at[pl.ds(it * CC + 128, CC - 128)]],
            ctx_buf.at[pl.ds(128, CC - 128)], sem)
        pltpu.async_copy(
            tgt_tab_hbm.at[tn_row_v.at[pl.ds(it * CT, CT)]], tn_buf, sem)

    def wait(ctx_buf, tn_buf, sem):
        pltpu.make_async_copy(ctx_tab_hbm.at[pl.ds(0, CC)], ctx_buf,
                              sem).wait()
        pltpu.make_async_copy(tgt_tab_hbm.at[pl.ds(0, CT)], tn_buf,
                              sem).wait()

    def compute(it, ctx_buf, tn_buf):
        e = iota16
        ctx_r = [e * CTX + j for j in range(CTX)]
        ctx_p = [plsc.load_gather(ctx_par_v, [e * CTX + (it * CC + j)])
                 for j in range(CTX)]
        tn_r = [e * TN + k for k in range(TN)]
        tn_p = [plsc.load_gather(tn_par_v, [e * TN + (it * CT + k)])
                for k in range(TN)]

        zero = jnp.zeros((16,), jnp.float32)

        @plsc.parallel_loop(0, DIM, carry=(zero, (zero,) * NEG), unroll=4)
        def d_loop(d, carry):
            pos, negs = carry
            col = (jnp.full((16,), d, jnp.int32) + iota16) & (DIM - 1)
            m = _tree_sum([plsc.load_gather(ctx_buf, [ctx_r[j],
                                                      ctx_p[j] + col])
                           for j in range(CTX)]) * (1.0 / CTX)
            t = plsc.load_gather(tn_buf, [tn_r[0], tn_p[0] + col])
            pos = pos + t * m
            negs = tuple(
                negs[k] + plsc.load_gather(
                    tn_buf, [tn_r[k + 1], tn_p[k + 1] + col]) * m
                for k in range(NEG))
            return pos, negs

        pos, negs = d_loop
        plsc.store_scatter(pos_v, [it * C + iota16], pos)
        for k in range(NEG):
            plsc.store_scatter(neg_v, [k * BPW + it * C + iota16], negs[k])

    fire(0, ctx_a, tn_a, sem_a)

    def pair_body(i, carry):
        it0 = 2 * i
        fire(it0 + 1, ctx_b, tn_b, sem_b)
        wait(ctx_a, tn_a, sem_a)
        compute(it0, ctx_a, tn_a)

        @pl.when(i < NPAIR - 1)
        def _():
            fire(it0 + 2, ctx_a, tn_a, sem_a)

        wait(ctx_b, tn_b, sem_b)
        compute(it0 + 1, ctx_b, tn_b)
        return carry

    lax.fori_loop(0, NPAIR, pair_body, 0)

    pltpu.sync_copy(pos_v, out_pos_hbm.at[pl.ds(base, BPW)])
    for k in range(NEG):
        pltpu.sync_copy(neg_v.at[pl.ds(k * BPW, BPW)],
                        out_neg_hbm.at[pl.ds(k * B + base, BPW)])


def _sc_scores(ctx_row, ctx_par, tn_row, tn_par, ctx_tab2, tgt_tab2):
    mesh = plsc.VectorSubcoreMesh(core_axis_name="c", subcore_axis_name="s")
    return pl.kernel(
        _sc_body,
        out_type=(jax.ShapeDtypeStruct((B,), jnp.float32),
                  jax.ShapeDtypeStruct((NEG * B,), jnp.float32)),
        mesh=mesh,
        compiler_params=pltpu.CompilerParams(needs_layout_passes=False),
        scratch_types=[
            pltpu.VMEM((BPW * CTX,), jnp.int32),
            pltpu.VMEM((BPW * CTX,), jnp.int32),
            pltpu.VMEM((BPW * TN,), jnp.int32),
            pltpu.VMEM((BPW * TN,), jnp.int32),
            pltpu.VMEM((CC, W), jnp.float32),
            pltpu.VMEM((CC, W), jnp.float32),
            pltpu.VMEM((CT, W), jnp.float32),
            pltpu.VMEM((CT, W), jnp.float32),
            pltpu.VMEM((BPW,), jnp.float32),
            pltpu.VMEM((NEG * BPW,), jnp.float32),
            pltpu.SemaphoreType.DMA,
            pltpu.SemaphoreType.DMA,
        ],
    )(ctx_row, ctx_par, tn_row, tn_par, ctx_tab2, tgt_tab2)


def _loss_body(pos_ref, neg_ref, out_ref):
    s = jnp.sum(jax.nn.log_sigmoid(pos_ref[...]))
    s = s + jnp.sum(jax.nn.log_sigmoid(-neg_ref[...]))
    out_ref[0, 0] = -s / B


def _tc_loss(pos2d, neg2d):
    return pl.pallas_call(
        _loss_body,
        out_shape=jax.ShapeDtypeStruct((1, 1), jnp.float32),
        out_specs=pl.BlockSpec(memory_space=pltpu.SMEM),
    )(pos2d, neg2d)


def kernel(context_indices, target_indices, negative_indices, context_table,
           target_table):
    ctx_idx = context_indices.astype(jnp.int32).reshape(B * CTX)
    tn_idx = jnp.concatenate(
        [target_indices.astype(jnp.int32).reshape(B, 1),
         negative_indices.astype(jnp.int32)], axis=1).reshape(B * TN)
    ctx_row = ctx_idx >> 1
    ctx_par = (ctx_idx & 1) * DIM
    tn_row = tn_idx >> 1
    tn_par = (tn_idx & 1) * DIM
    ctx_tab2 = context_table.reshape(V2, W)
    tgt_tab2 = target_table.reshape(V2, W)
    pos, neg = _sc_scores(ctx_row, ctx_par, tn_row, tn_par, ctx_tab2,
                          tgt_tab2)
    loss = _tc_loss(pos.reshape(128, 128), neg.reshape(NEG * 128, 128))
    return loss[0, 0]


# speedup vs baseline: 1.2717x; 1.0688x over previous
import jax
import jax.numpy as jnp
from jax import lax
from jax.experimental import pallas as pl
from jax.experimental.pallas import tpu as pltpu
from jax.experimental.pallas import tpu_sc as plsc

VOCAB = 1000000
DIM = 64
B = 16384
CTX = 10
NEG = 5
TN = NEG + 1
W = 128
V2 = VOCAB // 2

NC = 2
NS = 16
NW = NC * NS
BPW = B // NW
C = 16
NCHUNK = BPW // C
NPAIR = NCHUNK // 2
CC = C * CTX
CT = C * TN


def _tree_sum(vals):
    while len(vals) > 1:
        nxt = [vals[i] + vals[i + 1] for i in range(0, len(vals) - 1, 2)]
        if len(vals) % 2:
            nxt.append(vals[-1])
        vals = nxt
    return vals[0]


def _sc_body(ctx_row_hbm, ctx_par_hbm, tn_row_hbm, tn_par_hbm,
             ctx_tab_hbm, tgt_tab_hbm,
             out_pos_hbm, out_neg_hbm,
             ctx_row_v, ctx_par_v, tn_row_v, tn_par_v,
             ctx_a, ctx_b, tn_a, tn_b, pos_v, neg_v, sem_a, sem_b):
    wid = lax.axis_index("s") * NC + lax.axis_index("c")
    base = wid * BPW

    pltpu.sync_copy(ctx_row_hbm.at[pl.ds(base * CTX, BPW * CTX)], ctx_row_v)
    pltpu.sync_copy(ctx_par_hbm.at[pl.ds(base * CTX, BPW * CTX)], ctx_par_v)
    pltpu.sync_copy(tn_row_hbm.at[pl.ds(base * TN, BPW * TN)], tn_row_v)
    pltpu.sync_copy(tn_par_hbm.at[pl.ds(base * TN, BPW * TN)], tn_par_v)

    iota16 = lax.iota(jnp.int32, 16)

    def fire(it, ctx_buf, tn_buf, sem):
        pltpu.async_copy(
            ctx_tab_hbm.at[ctx_row_v.at[pl.ds(it * CC, 128)]],
            ctx_buf.at[pl.ds(0, 128)], sem)
        pltpu.async_copy(
            ctx_tab_hbm.at[ctx_row_v.at[pl.ds(it * CC + 128, CC - 128)]],
            ctx_buf.at[pl.ds(128, CC - 128)], sem)
        pltpu.async_copy(
            tgt_tab_hbm.at[tn_row_v.at[pl.ds(it * CT, CT)]], tn_buf, sem)

    def wait(ctx_buf, tn_buf, sem):
        pltpu.make_async_copy(ctx_tab_hbm.at[pl.ds(0, CC)], ctx_buf,
                              sem).wait()
        pltpu.make_async_copy(tgt_tab_hbm.at[pl.ds(0, CT)], tn_buf,
                              sem).wait()

    def compute(it, ctx_buf, tn_buf):
        e = iota16
        ctx_r = [e * CTX + j for j in range(CTX)]
        ctx_p = [plsc.load_gather(ctx_par_v, [e * CTX + (it * CC + j)])
                 for j in range(CTX)]
        tn_r = [e * TN + k for k in range(TN)]
        tn_p = [plsc.load_gather(tn_par_v, [e * TN + (it * CT + k)])
                for k in range(TN)]

        zero = jnp.zeros((16,), jnp.float32)

        @plsc.parallel_loop(0, DIM, carry=(zero, (zero,) * NEG), unroll=4)
        def d_loop(d, carry):
            pos, negs = carry
            col = (jnp.full((16,), d, jnp.int32) + iota16) & (DIM - 1)
            m = _tree_sum([plsc.load_gather(ctx_buf, [ctx_r[j],
                                                      ctx_p[j] + col])
                           for j in range(CTX)]) * (1.0 / CTX)
            t = plsc.load_gather(tn_buf, [tn_r[0], tn_p[0] + col])
            pos = pos + t * m
            negs = tuple(
                negs[k] + plsc.load_gather(
                    tn_buf, [tn_r[k + 1], tn_p[k + 1] + col]) * m
                for k in range(NEG))
            return pos, negs

        pos, negs = d_loop
        plsc.store_scatter(pos_v, [it * C + iota16], pos)
        for k in range(NEG):
            plsc.store_scatter(neg_v, [k * BPW + it * C + iota16], negs[k])

    fire(0, ctx_a, tn_a, sem_a)

    def pair_body(i, carry):
        it0 = 2 * i
        fire(it0 + 1, ctx_b, tn_b, sem_b)
        wait(ctx_a, tn_a, sem_a)
        compute(it0, ctx_a, tn_a)

        @pl.when(i < NPAIR - 1)
        def _():
            fire(it0 + 2, ctx_a, tn_a, sem_a)

        wait(ctx_b, tn_b, sem_b)
        compute(it0 + 1, ctx_b, tn_b)
        return carry

    lax.fori_loop(0, NPAIR, pair_body, 0)

    pltpu.sync_copy(pos_v, out_pos_hbm.at[pl.ds(base, BPW)])
    for k in range(NEG):
        pltpu.sync_copy(neg_v.at[pl.ds(k * BPW, BPW)],
                        out_neg_hbm.at[pl.ds(k * B + base, BPW)])


def _sc_scores(ctx_row, ctx_par, tn_row, tn_par, ctx_tab2, tgt_tab2):
    mesh = plsc.VectorSubcoreMesh(core_axis_name="c", subcore_axis_name="s")
    return pl.kernel(
        _sc_body,
        out_type=(jax.ShapeDtypeStruct((B,), jnp.float32),
                  jax.ShapeDtypeStruct((NEG * B,), jnp.float32)),
        mesh=mesh,
        compiler_params=pltpu.CompilerParams(needs_layout_passes=False),
        scratch_types=[
            pltpu.VMEM((BPW * CTX,), jnp.int32),
            pltpu.VMEM((BPW * CTX,), jnp.int32),
            pltpu.VMEM((BPW * TN,), jnp.int32),
            pltpu.VMEM((BPW * TN,), jnp.int32),
            pltpu.VMEM((CC, W), jnp.float32),
            pltpu.VMEM((CC, W), jnp.float32),
            pltpu.VMEM((CT, W), jnp.float32),
            pltpu.VMEM((CT, W), jnp.float32),
            pltpu.VMEM((BPW,), jnp.float32),
            pltpu.VMEM((NEG * BPW,), jnp.float32),
            pltpu.SemaphoreType.DMA,
            pltpu.SemaphoreType.DMA,
        ],
    )(ctx_row, ctx_par, tn_row, tn_par, ctx_tab2, tgt_tab2)


def _loss_body(pos_ref, neg_ref, out_ref):
    s = jnp.sum(jax.nn.log_sigmoid(pos_ref[...]))
    s = s + jnp.sum(jax.nn.log_sigmoid(-neg_ref[...]))
    out_ref[0, 0] = -s / B


def _tc_loss(pos2d, neg2d):
    return pl.pallas_call(
        _loss_body,
        out_shape=jax.ShapeDtypeStruct((1, 1), jnp.float32),
        out_specs=pl.BlockSpec(memory_space=pltpu.SMEM),
    )(pos2d, neg2d)


def kernel(context_indices, target_indices, negative_indices, context_table,
           target_table):
    ctx_idx = context_indices.astype(jnp.int32).reshape(B * CTX)
    tn_idx = jnp.concatenate(
        [target_indices.astype(jnp.int32).reshape(B, 1),
         negative_indices.astype(jnp.int32)], axis=1).reshape(B * TN)
    ctx_row = ctx_idx
    ctx_par = jnp.zeros_like(ctx_idx)
    tn_row = tn_idx
    tn_par = jnp.zeros_like(tn_idx)
    ctx_tab2 = jnp.pad(context_table, ((0, 0), (0, W - DIM)))
    tgt_tab2 = jnp.pad(target_table, ((0, 0), (0, W - DIM)))
    pos, neg = _sc_scores(ctx_row, ctx_par, tn_row, tn_par, ctx_tab2,
                          tgt_tab2)
    loss = _tc_loss(pos.reshape(128, 128), neg.reshape(NEG * 128, 128))
    return loss[0, 0]
